# baseline (speedup 1.0000x reference)
import numpy as np
import jax
import jax.numpy as jnp
from jax.experimental import pallas as pl
from jax.experimental.pallas import tpu as pltpu

_B, _A, _ATTRS = 16, 3, 6
_D = _H = _W = 48
_HW = _H * _W
_DHW = _D * _HW
_NROWS = _DHW // 128
_STRIDE = 2.0
_SBLK = 96


def _decode_kernel(in_ref, out_ref):
    a = pl.program_id(1)
    si = pl.program_id(2)

    c = in_ref[0, 0]

    base = si * (_SBLK * 128)
    g = (
        base
        + 128 * jax.lax.broadcasted_iota(jnp.int32, (_SBLK, 128), 0)
        + jax.lax.broadcasted_iota(jnp.int32, (_SBLK, 128), 1)
    )
    d = g // _HW
    hw = g - d * _HW
    h = hw // _W
    w = hw - h * _W

    aw = jnp.where(a == 0, 4.0, jnp.where(a == 1, 8.0, 16.0))
    vals = [
        (jax.nn.sigmoid(c[0]) + w.astype(jnp.float32)) * _STRIDE,
        (jax.nn.sigmoid(c[1]) + h.astype(jnp.float32)) * _STRIDE,
        (jax.nn.sigmoid(c[2]) + d.astype(jnp.float32)) * _STRIDE,
        jnp.exp(c[3]) * aw,
        jax.nn.sigmoid(c[4]),
        jax.nn.sigmoid(c[5]),
    ]

    lam = jax.lax.broadcasted_iota(jnp.int32, (_SBLK, 128), 1)
    tiles = []
    for u in range(_ATTRS):
        idx = (128 * u + lam) // 6
        jsel = (2 * u + lam) % 6
        acc = jnp.take_along_axis(vals[5], idx, axis=1)
        for j in range(4, -1, -1):
            gj = jnp.take_along_axis(vals[j], idx, axis=1)
            acc = jnp.where(jsel == j, gj, acc)
        tiles.append(acc)
    out_ref[0, 0] = jnp.concatenate(tiles, axis=-1)


def kernel(inp):
    b = inp.shape[0]
    view = inp.reshape(b, _A, _ATTRS, _NROWS, 128)
    out = pl.pallas_call(
        _decode_kernel,
        grid=(b, _A, _NROWS // _SBLK),
        in_specs=[
            pl.BlockSpec(
                (1, 1, _ATTRS, _SBLK, 128),
                lambda bi, ai, si: (bi, ai, 0, si, 0),
            )
        ],
        out_specs=pl.BlockSpec(
            (1, 1, _SBLK, _ATTRS * 128),
            lambda bi, ai, si: (bi, ai, si, 0),
        ),
        out_shape=jax.ShapeDtypeStruct(
            (b, _A, _NROWS, _ATTRS * 128), jnp.float32
        ),
        compiler_params=pltpu.CompilerParams(
            dimension_semantics=("parallel", "parallel", "arbitrary"),
            vmem_limit_bytes=56 * 1024 * 1024,
        ),
    )(view)
    return out.reshape(b, _A * _DHW, _ATTRS)


# speedup vs baseline: 1.7589x; 1.7589x over previous
import jax
import jax.numpy as jnp
from jax.experimental import pallas as pl
from jax.experimental.pallas import tpu as pltpu

_B, _A, _ATTRS = 16, 3, 6
_D = _H = _W = 48
_STRIDE = 2.0
_DBLK = 8
_HBLK = 16


def _decode_kernel(in_ref, out_ref):
    a = pl.program_id(1)
    di = pl.program_id(2)
    hi = pl.program_id(3)

    c = in_ref[0, 0]

    shape = (_DBLK, _HBLK, _W)
    gx = jax.lax.broadcasted_iota(jnp.int32, shape, 2).astype(jnp.float32)
    gy = (
        jax.lax.broadcasted_iota(jnp.int32, shape, 1) + hi * _HBLK
    ).astype(jnp.float32)
    gz = (
        jax.lax.broadcasted_iota(jnp.int32, shape, 0) + di * _DBLK
    ).astype(jnp.float32)

    aw = jnp.where(a == 0, 4.0, jnp.where(a == 1, 8.0, 16.0))
    vals = [
        (jax.nn.sigmoid(c[0]) + gx) * _STRIDE,
        (jax.nn.sigmoid(c[1]) + gy) * _STRIDE,
        (jax.nn.sigmoid(c[2]) + gz) * _STRIDE,
        jnp.exp(c[3]) * aw,
        jax.nn.sigmoid(c[4]),
        jax.nn.sigmoid(c[5]),
    ]

    s = jnp.stack(vals, axis=2)
    out_ref[0, 0] = jnp.swapaxes(s, 2, 3)


def kernel(inp):
    b = inp.shape[0]
    view = inp.reshape(b, _A, _ATTRS, _D, _H, _W)
    out = pl.pallas_call(
        _decode_kernel,
        grid=(b, _A, _D // _DBLK, _H // _HBLK),
        in_specs=[
            pl.BlockSpec(
                (1, 1, _ATTRS, _DBLK, _HBLK, _W),
                lambda bi, ai, di, hi: (bi, ai, 0, di, hi, 0),
            )
        ],
        out_specs=pl.BlockSpec(
            (1, 1, _DBLK, _HBLK, _W, _ATTRS),
            lambda bi, ai, di, hi: (bi, ai, di, hi, 0, 0),
        ),
        out_shape=jax.ShapeDtypeStruct(
            (b, _A, _D, _H, _W, _ATTRS), jnp.float32
        ),
        compiler_params=pltpu.CompilerParams(
            dimension_semantics=("parallel", "parallel", "arbitrary", "arbitrary"),
            vmem_limit_bytes=56 * 1024 * 1024,
        ),
    )(view)
    return out.reshape(b, _A * _D * _H * _W, _ATTRS)


# speedup vs baseline: 3.3442x; 1.9013x over previous
import jax
import jax.numpy as jnp
from jax.experimental import pallas as pl
from jax.experimental.pallas import tpu as pltpu

_B, _A, _ATTRS = 16, 3, 6
_D = _H = _W = 48
_HW = _H * _W
_DHW = _D * _HW
_STRIDE = 2.0
_NBLK = 9216
_NCH = _DHW // _NBLK


def _decode_kernel(in_ref, out_ref):
    j = pl.program_id(0)
    a = pl.program_id(1)
    ni = pl.program_id(2)

    v = in_ref[0]
    n = ni * _NBLK + jax.lax.broadcasted_iota(jnp.int32, (_B, _NBLK), 1)

    @pl.when(j == 0)
    def _():
        gx = (n % _W).astype(jnp.float32)
        out_ref[0] = (jax.nn.sigmoid(v) + gx) * _STRIDE

    @pl.when(j == 1)
    def _():
        gy = ((n // _W) % _H).astype(jnp.float32)
        out_ref[0] = (jax.nn.sigmoid(v) + gy) * _STRIDE

    @pl.when(j == 2)
    def _():
        gz = (n // _HW).astype(jnp.float32)
        out_ref[0] = (jax.nn.sigmoid(v) + gz) * _STRIDE

    @pl.when(j == 3)
    def _():
        aw = jnp.where(a == 0, 4.0, jnp.where(a == 1, 8.0, 16.0))
        out_ref[0] = jnp.exp(v) * aw

    @pl.when(j >= 4)
    def _():
        out_ref[0] = jax.nn.sigmoid(v)


def kernel(inp):
    b = inp.shape[0]
    src = inp.reshape(b, _A * _ATTRS, _DHW).transpose(1, 0, 2)
    out = pl.pallas_call(
        _decode_kernel,
        grid=(_ATTRS, _A, _NCH),
        in_specs=[
            pl.BlockSpec(
                (1, b, _NBLK),
                lambda ji, ai, ni: (ai * _ATTRS + ji, 0, ni),
            )
        ],
        out_specs=pl.BlockSpec(
            (1, b, _NBLK),
            lambda ji, ai, ni: (ji, 0, ai * _NCH + ni),
        ),
        out_shape=jax.ShapeDtypeStruct((_ATTRS, b, _A * _DHW), jnp.float32),
        compiler_params=pltpu.CompilerParams(
            dimension_semantics=("parallel", "parallel", "arbitrary"),
            vmem_limit_bytes=56 * 1024 * 1024,
        ),
    )(src)
    return out.transpose(1, 2, 0)


# speedup vs baseline: 3.6533x; 1.0924x over previous
import jax
import jax.numpy as jnp
from jax.experimental import pallas as pl
from jax.experimental.pallas import tpu as pltpu

_B, _A, _ATTRS = 16, 3, 6
_D = _H = _W = 48
_HW = _H * _W
_DHW = _D * _HW
_STRIDE = 2.0
_NBLK = 27648
_NCH = _DHW // _NBLK


def _decode_kernel(in_ref, out_ref):
    j = pl.program_id(0)
    a = pl.program_id(1)
    ni = pl.program_id(2)

    v = in_ref[0]
    n = ni * _NBLK + jax.lax.broadcasted_iota(jnp.int32, (_B, _NBLK), 1)

    @pl.when(j == 0)
    def _():
        gx = (n % _W).astype(jnp.float32)
        out_ref[0] = (jax.nn.sigmoid(v) + gx) * _STRIDE

    @pl.when(j == 1)
    def _():
        gy = ((n // _W) % _H).astype(jnp.float32)
        out_ref[0] = (jax.nn.sigmoid(v) + gy) * _STRIDE

    @pl.when(j == 2)
    def _():
        gz = (n // _HW).astype(jnp.float32)
        out_ref[0] = (jax.nn.sigmoid(v) + gz) * _STRIDE

    @pl.when(j == 3)
    def _():
        aw = jnp.where(a == 0, 4.0, jnp.where(a == 1, 8.0, 16.0))
        out_ref[0] = jnp.exp(v) * aw

    @pl.when(j >= 4)
    def _():
        out_ref[0] = jax.nn.sigmoid(v)


def kernel(inp):
    b = inp.shape[0]
    src = inp.reshape(b, _A * _ATTRS, _DHW).transpose(1, 0, 2)
    out = pl.pallas_call(
        _decode_kernel,
        grid=(_ATTRS, _A, _NCH),
        in_specs=[
            pl.BlockSpec(
                (1, b, _NBLK),
                lambda ji, ai, ni: (ai * _ATTRS + ji, 0, ni),
            )
        ],
        out_specs=pl.BlockSpec(
            (1, b, _NBLK),
            lambda ji, ai, ni: (ji, 0, ai * _NCH + ni),
        ),
        out_shape=jax.ShapeDtypeStruct((_ATTRS, b, _A * _DHW), jnp.float32),
        compiler_params=pltpu.CompilerParams(
            dimension_semantics=("parallel", "parallel", "arbitrary"),
            vmem_limit_bytes=56 * 1024 * 1024,
        ),
    )(src)
    return out.transpose(1, 2, 0)


# speedup vs baseline: 3.7401x; 1.0238x over previous
import jax
import jax.numpy as jnp
from jax.experimental import pallas as pl
from jax.experimental.pallas import tpu as pltpu

_B, _A, _ATTRS = 16, 3, 6
_D = _H = _W = 48
_HW = _H * _W
_DHW = _D * _HW
_STRIDE = 2.0
_NBLK = 55296
_NCH = _DHW // _NBLK


def _decode_kernel(in_ref, out_ref):
    j = pl.program_id(0)
    a = pl.program_id(1)
    ni = pl.program_id(2)

    v = in_ref[0]
    n = ni * _NBLK + jax.lax.broadcasted_iota(jnp.int32, (_B, _NBLK), 1)

    @pl.when(j == 0)
    def _():
        gx = (n % _W).astype(jnp.float32)
        out_ref[0] = (jax.nn.sigmoid(v) + gx) * _STRIDE

    @pl.when(j == 1)
    def _():
        gy = ((n // _W) % _H).astype(jnp.float32)
        out_ref[0] = (jax.nn.sigmoid(v) + gy) * _STRIDE

    @pl.when(j == 2)
    def _():
        gz = (n // _HW).astype(jnp.float32)
        out_ref[0] = (jax.nn.sigmoid(v) + gz) * _STRIDE

    @pl.when(j == 3)
    def _():
        aw = jnp.where(a == 0, 4.0, jnp.where(a == 1, 8.0, 16.0))
        out_ref[0] = jnp.exp(v) * aw

    @pl.when(j >= 4)
    def _():
        out_ref[0] = jax.nn.sigmoid(v)


def kernel(inp):
    b = inp.shape[0]
    src = inp.reshape(b, _A * _ATTRS, _DHW).transpose(1, 0, 2)
    out = pl.pallas_call(
        _decode_kernel,
        grid=(_ATTRS, _A, _NCH),
        in_specs=[
            pl.BlockSpec(
                (1, b, _NBLK),
                lambda ji, ai, ni: (ai * _ATTRS + ji, 0, ni),
            )
        ],
        out_specs=pl.BlockSpec(
            (1, b, _NBLK),
            lambda ji, ai, ni: (ji, 0, ai * _NCH + ni),
        ),
        out_shape=jax.ShapeDtypeStruct((_ATTRS, b, _A * _DHW), jnp.float32),
        compiler_params=pltpu.CompilerParams(
            dimension_semantics=("parallel", "parallel", "arbitrary"),
            vmem_limit_bytes=56 * 1024 * 1024,
        ),
    )(src)
    return out.transpose(1, 2, 0)


# speedup vs baseline: 4.2260x; 1.1299x over previous
import jax
import jax.numpy as jnp
from jax.experimental import pallas as pl
from jax.experimental.pallas import tpu as pltpu

_B, _A, _ATTRS = 16, 3, 6
_D = _H = _W = 48
_HW = _H * _W
_DHW = _D * _HW
_STRIDE = 2.0
_NBLK = 55296
_NCH = _DHW // _NBLK


def _decode_kernel(in_ref, out_ref):
    j = pl.program_id(0)
    a = pl.program_id(1)
    ni = pl.program_id(2)

    v = in_ref[0]
    n = ni * _NBLK + jax.lax.broadcasted_iota(jnp.int32, (_B, _NBLK), 1)

    @pl.when(j == 0)
    def _():
        gx = (n % _W).astype(jnp.float32)
        out_ref[0] = (jax.nn.sigmoid(v) + gx) * _STRIDE

    @pl.when(j == 1)
    def _():
        gy = ((n // _W) % _H).astype(jnp.float32)
        out_ref[0] = (jax.nn.sigmoid(v) + gy) * _STRIDE

    @pl.when(j == 2)
    def _():
        gz = (n // _HW).astype(jnp.float32)
        out_ref[0] = (jax.nn.sigmoid(v) + gz) * _STRIDE

    @pl.when(j == 3)
    def _():
        aw = jnp.where(a == 0, 4.0, jnp.where(a == 1, 8.0, 16.0))
        out_ref[0] = jnp.exp(v) * aw

    @pl.when(j >= 4)
    def _():
        out_ref[0] = jax.nn.sigmoid(v)


def kernel(inp):
    b = inp.shape[0]
    src = jax.lax.reshape(
        inp, (_A * _ATTRS, b, _DHW), dimensions=(1, 0, 2, 3, 4)
    )
    out = pl.pallas_call(
        _decode_kernel,
        grid=(_ATTRS, _A, _NCH),
        in_specs=[
            pl.BlockSpec(
                (1, b, _NBLK),
                lambda ji, ai, ni: (ai * _ATTRS + ji, 0, ni),
            )
        ],
        out_specs=pl.BlockSpec(
            (1, b, _NBLK),
            lambda ji, ai, ni: (ji, 0, ai * _NCH + ni),
        ),
        out_shape=jax.ShapeDtypeStruct((_ATTRS, b, _A * _DHW), jnp.float32),
        compiler_params=pltpu.CompilerParams(
            dimension_semantics=("parallel", "parallel", "arbitrary"),
            vmem_limit_bytes=56 * 1024 * 1024,
        ),
    )(src)
    return out.transpose(1, 2, 0)


# speedup vs baseline: 4.3208x; 1.0224x over previous
import jax
import jax.numpy as jnp
from jax.experimental import pallas as pl
from jax.experimental.pallas import tpu as pltpu

_B, _A, _ATTRS = 16, 3, 6
_D = _H = _W = 48
_HW = _H * _W
_DHW = _D * _HW
_STRIDE = 2.0
_NBLK = 110592
_NCH = _DHW // _NBLK


def _decode_kernel(in_ref, out_ref):
    j = pl.program_id(0)
    a = pl.program_id(1)
    ni = pl.program_id(2)

    v = in_ref[0]
    n = ni * _NBLK + jax.lax.broadcasted_iota(jnp.int32, (_B, _NBLK), 1)

    @pl.when(j == 0)
    def _():
        gx = (n % _W).astype(jnp.float32)
        out_ref[0] = (jax.nn.sigmoid(v) + gx) * _STRIDE

    @pl.when(j == 1)
    def _():
        gy = ((n // _W) % _H).astype(jnp.float32)
        out_ref[0] = (jax.nn.sigmoid(v) + gy) * _STRIDE

    @pl.when(j == 2)
    def _():
        gz = (n // _HW).astype(jnp.float32)
        out_ref[0] = (jax.nn.sigmoid(v) + gz) * _STRIDE

    @pl.when(j == 3)
    def _():
        aw = jnp.where(a == 0, 4.0, jnp.where(a == 1, 8.0, 16.0))
        out_ref[0] = jnp.exp(v) * aw

    @pl.when(j >= 4)
    def _():
        out_ref[0] = jax.nn.sigmoid(v)


def kernel(inp):
    b = inp.shape[0]
    src = jax.lax.reshape(
        inp, (_A * _ATTRS, b, _DHW), dimensions=(1, 0, 2, 3, 4)
    )
    out = pl.pallas_call(
        _decode_kernel,
        grid=(_ATTRS, _A, _NCH),
        in_specs=[
            pl.BlockSpec(
                (1, b, _NBLK),
                lambda ji, ai, ni: (ai * _ATTRS + ji, 0, ni),
            )
        ],
        out_specs=pl.BlockSpec(
            (1, b, _NBLK),
            lambda ji, ai, ni: (ji, 0, ai * _NCH + ni),
        ),
        out_shape=jax.ShapeDtypeStruct((_ATTRS, b, _A * _DHW), jnp.float32),
        compiler_params=pltpu.CompilerParams(
            dimension_semantics=("parallel", "parallel", "arbitrary"),
            vmem_limit_bytes=56 * 1024 * 1024,
        ),
    )(src)
    return out.transpose(1, 2, 0)
